# baseline (speedup 1.0000x reference)
import functools

import jax
import jax.numpy as jnp
from jax import lax
from jax.experimental import pallas as pl
from jax.experimental.pallas import tpu as pltpu
from jax.experimental.pallas import tpu_sc as plsc

N_NODES = 10000
N_MAP = 128
E = 320000
ROW_BLK = 2000
_EPS = 1e-5

NT = 32
NPT = 320
BUF_ROWS = 328
DUMMY = NPT
C1 = 10000
LW = E + C1 + 512
G = 512


def _gn(x, g, b):
    mu = jnp.mean(x, axis=1, keepdims=True)
    var = jnp.mean((x - mu) ** 2, axis=1, keepdims=True)
    return (x - mu) * jax.lax.rsqrt(var + _EPS) * g + b


def _in_mlp_body(x_ref, w1_ref, g1_ref, b1_ref, w2_ref, g2_ref, b2_ref,
                 wt_ref, gt_ref, bt_ref, o_ref):
    x = x_ref[...]
    h = jax.nn.relu(_gn(jnp.dot(x, w1_ref[...],
                                preferred_element_type=jnp.float32),
                        g1_ref[...], b1_ref[...]))
    h2 = _gn(jnp.dot(h, w2_ref[...], preferred_element_type=jnp.float32),
             g2_ref[...], b2_ref[...])
    t = _gn(jnp.dot(x, wt_ref[...], preferred_element_type=jnp.float32),
            gt_ref[...], bt_ref[...])
    o_ref[...] = jax.nn.relu(h2 + t)


def _input_mlp(feats, w1, g1, b1, w2, g2, b2, wt, gt, bt):
    n = feats.shape[0]
    row_spec = pl.BlockSpec((ROW_BLK, feats.shape[1]), lambda i: (i, 0))
    full = lambda a: pl.BlockSpec(a.shape, lambda i: (0,) * a.ndim)
    return pl.pallas_call(
        _in_mlp_body,
        grid=(n // ROW_BLK,),
        in_specs=[row_spec] + [full(a) for a in (w1, g1, b1, w2, g2, b2, wt, gt, bt)],
        out_specs=pl.BlockSpec((ROW_BLK, N_MAP), lambda i: (i, 0)),
        out_shape=jax.ShapeDtypeStruct((n, N_MAP), jnp.float32),
    )(feats, w1, g1, b1, w2, g2, b2, wt, gt, bt)


def _pre_body(x_ref, w_ref, g_ref, b_ref, o_ref):
    o_ref[...] = jax.nn.relu(
        _gn(jnp.dot(x_ref[...], w_ref[...], preferred_element_type=jnp.float32),
            g_ref[...], b_ref[...]))


def _pre(feat, w, g, b):
    n = feat.shape[0]
    full = lambda a: pl.BlockSpec(a.shape, lambda i: (0,) * a.ndim)
    return pl.pallas_call(
        _pre_body,
        grid=(n // ROW_BLK,),
        in_specs=[pl.BlockSpec((ROW_BLK, N_MAP), lambda i: (i, 0)),
                  full(w), full(g), full(b)],
        out_specs=pl.BlockSpec((ROW_BLK, N_MAP), lambda i: (i, 0)),
        out_shape=jax.ShapeDtypeStruct((n, N_MAP), jnp.float32),
    )(feat, w, g, b)


def _post_body(feat_ref, agg_ref, wa_ref, wb_ref, g2_ref, b2_ref,
               wl_ref, gl_ref, bl_ref, o_ref):
    feat = feat_ref[...]
    x = (jnp.dot(feat, wa_ref[...], preferred_element_type=jnp.float32)
         + jnp.dot(agg_ref[...], wb_ref[...], preferred_element_type=jnp.float32))
    x = jax.nn.relu(_gn(x, g2_ref[...], b2_ref[...]))
    x = _gn(jnp.dot(x, wl_ref[...], preferred_element_type=jnp.float32),
            gl_ref[...], bl_ref[...])
    o_ref[...] = jax.nn.relu(x + feat)


def _post(feat, agg, w2, g2, b2, wl, gl, bl):
    n = feat.shape[0]
    wa, wb = w2[:N_MAP], w2[N_MAP:]
    full = lambda a: pl.BlockSpec(a.shape, lambda i: (0,) * a.ndim)
    row = pl.BlockSpec((ROW_BLK, N_MAP), lambda i: (i, 0))
    return pl.pallas_call(
        _post_body,
        grid=(n // ROW_BLK,),
        in_specs=[row, row, full(wa), full(wb), full(g2), full(b2),
                  full(wl), full(gl), full(bl)],
        out_specs=row,
        out_shape=jax.ShapeDtypeStruct((n, N_MAP), jnp.float32),
    )(feat, agg, wa, wb, g2, b2, wl, gl, bl)


def _wid():
    return lax.axis_index("s") * 2 + lax.axis_index("c")


_MESH = plsc.VectorSubcoreMesh(core_axis_name="c", subcore_axis_name="s")


@functools.partial(
    pl.kernel,
    mesh=_MESH,
    out_type=[jax.ShapeDtypeStruct((NT * LW,), jnp.int32),
              jax.ShapeDtypeStruct((NT * 16,), jnp.int32)],
    scratch_types=[pltpu.VMEM((C1,), jnp.int32),
                   pltpu.VMEM((C1,), jnp.int32),
                   pltpu.VMEM((C1 + 16,), jnp.int32),
                   pltpu.VMEM((16,), jnp.int32)],
)
def _phase1(u_hbm, v_hbm, list_hbm, cnt_hbm, ubuf, vbuf, stage, cstage):
    w = _wid()
    lo = w * NPT
    hi = lo + NPT
    zero16 = jnp.zeros((16,), jnp.int32)

    def _fill(i, _):
        stage[pl.ds(i * 16, 16)] = zero16
        return 0
    lax.fori_loop(0, (C1 + 16) // 16, _fill, 0)

    def _vec(i, sc):
        vv = vbuf[pl.ds(i * 16, 16)]
        uu = ubuf[pl.ds(i * 16, 16)]
        m = (vv >= lo) & (vv < hi)
        mi = m.astype(jnp.int32)
        pk = (uu << 9) | (vv - lo + 1)
        stage[pl.ds(sc, 16)] = zero16
        pos = plsc.cumsum(mi) + (sc - 1)
        plsc.addupdate_scatter(stage, [pos], pk, mask=m)
        return pos[15] + 1

    cnt = 0
    for k in range(E // C1):
        pltpu.sync_copy(u_hbm.at[pl.ds(k * C1, C1)], ubuf)
        pltpu.sync_copy(v_hbm.at[pl.ds(k * C1, C1)], vbuf)
        sc = lax.fori_loop(0, C1 // 16, _vec, 0)
        scp = (sc + 15) & ~15
        off = pl.multiple_of(w * LW + cnt, 16)
        pltpu.sync_copy(stage.at[pl.ds(0, C1)], list_hbm.at[pl.ds(off, C1)])
        cnt = cnt + scp
    cstage[...] = jnp.full((16,), 0, jnp.int32) + cnt
    pltpu.sync_copy(cstage, cnt_hbm.at[pl.ds(pl.multiple_of(w * 16, 16), 16)])


@functools.partial(
    pl.kernel,
    mesh=_MESH,
    out_type=jax.ShapeDtypeStruct((NT * NPT, N_MAP), jnp.float32),
    scratch_types=[pltpu.VMEM((BUF_ROWS, N_MAP), jnp.float32),
                   pltpu.VMEM((G,), jnp.int32),
                   pltpu.VMEM((G,), jnp.int32),
                   pltpu.VMEM((G,), jnp.int32),
                   pltpu.VMEM((G, N_MAP), jnp.float32),
                   pltpu.VMEM((16,), jnp.int32),
                   pltpu.SemaphoreType.DMA],
)
def _phase2(ctx_hbm, list_hbm, cnt_hbm, agg_hbm,
            buf, pbuf, idx, nbb, rows, cbuf, sem):
    w = _wid()
    lo = w * NPT

    pltpu.sync_copy(cnt_hbm.at[pl.ds(pl.multiple_of(w * 16, 16), 16)], cbuf)
    cnt = cbuf[...][0]

    zero = jnp.zeros((16,), jnp.float32)

    def _zero(r, _):
        for c in range(N_MAP // 16):
            buf[r, pl.ds(c * 16, 16)] = zero
        return 0
    lax.fori_loop(0, BUF_ROWS, _zero, 0)

    nchunks = (cnt + G - 1) // G

    def _chunk(k, _):
        off = pl.multiple_of(w * LW + k * G, 16)
        pltpu.sync_copy(list_hbm.at[pl.ds(off, G)], pbuf)

        def _unpack(i, _):
            pk = pbuf[pl.ds(i * 16, 16)]
            iv = pk == 0
            spread = w * 16 + lax.iota(jnp.int32, 16)
            idx[pl.ds(i * 16, 16)] = jnp.where(iv, spread, pk >> 9)
            nbb[pl.ds(i * 16, 16)] = jnp.where(iv, NPT + 1, pk & 511) - 1
            return 0
        lax.fori_loop(0, G // 16, _unpack, 0)

        pltpu.async_copy(ctx_hbm.at[idx], rows, sem).wait()

        def _grp(i, _):
            nb16 = nbb[pl.ds(i * 16, 16)]
            for l in range(16):
                g = i * 16 + l
                nb = nb16[l]
                for c in range(N_MAP // 16):
                    sl = pl.ds(c * 16, 16)
                    buf[nb, sl] = jnp.maximum(buf[nb, sl], rows[g, sl])
            return 0
        lax.fori_loop(0, G // 16, _grp, 0)
        return 0

    lax.fori_loop(0, nchunks, _chunk, 0)
    pltpu.sync_copy(buf.at[pl.ds(0, NPT)], agg_hbm.at[pl.ds(lo, NPT)])


def kernel(feats, u, v, in_W1, in_g1, in_b1, in_W2, in_g2, in_b2, in_Wt,
           in_gt, in_bt, fc1_W, fc1_g, fc1_b, fc2_W, fc2_g, fc2_b,
           lin_W, lin_g, lin_b):
    feat = _input_mlp(feats, in_W1, in_g1, in_b1, in_W2, in_g2, in_b2,
                      in_Wt, in_gt, in_bt)
    n_scales = u.shape[0]
    n_blk = fc1_W.shape[0] // n_scales
    sorted_edges = []
    for i in range(n_scales):
        vs, us = jax.lax.sort((v[i], u[i]), num_keys=1)
        sorted_edges.append((us, vs))
    for j in range(n_blk):
        for i in range(n_scales):
            t = j * n_scales + i
            ctx = _pre(feat, fc1_W[t], fc1_g[t], fc1_b[t])
            us, vs = sorted_edges[i]
            gathered = jnp.take(ctx, us, axis=0)
            agg = jax.ops.segment_max(gathered, vs, num_segments=N_NODES,
                                      indices_are_sorted=True)
            agg = jnp.where(agg == -jnp.inf, 0.0, agg)
            feat = _post(feat, agg, fc2_W[t], fc2_g[t], fc2_b[t],
                         lin_W[t], lin_g[t], lin_b[t])
    return feat


# speedup vs baseline: 2.3315x; 2.3315x over previous
import functools

import jax
import jax.numpy as jnp
from jax import lax
from jax.experimental import pallas as pl
from jax.experimental.pallas import tpu as pltpu
from jax.experimental.pallas import tpu_sc as plsc

N_NODES = 10000
N_MAP = 128
E = 320000
ROW_BLK = 2000
_EPS = 1e-5

NT = 32
NPT = 320
BUF_ROWS = 328
G = 512
EP = E + G


def _gn(x, g, b):
    mu = jnp.mean(x, axis=1, keepdims=True)
    var = jnp.mean((x - mu) ** 2, axis=1, keepdims=True)
    return (x - mu) * jax.lax.rsqrt(var + _EPS) * g + b


def _in_mlp_body(x_ref, w1_ref, g1_ref, b1_ref, w2_ref, g2_ref, b2_ref,
                 wt_ref, gt_ref, bt_ref, o_ref):
    x = x_ref[...]
    h = jax.nn.relu(_gn(jnp.dot(x, w1_ref[...],
                                preferred_element_type=jnp.float32),
                        g1_ref[...], b1_ref[...]))
    h2 = _gn(jnp.dot(h, w2_ref[...], preferred_element_type=jnp.float32),
             g2_ref[...], b2_ref[...])
    t = _gn(jnp.dot(x, wt_ref[...], preferred_element_type=jnp.float32),
            gt_ref[...], bt_ref[...])
    o_ref[...] = jax.nn.relu(h2 + t)


def _input_mlp(feats, w1, g1, b1, w2, g2, b2, wt, gt, bt):
    n = feats.shape[0]
    row_spec = pl.BlockSpec((ROW_BLK, feats.shape[1]), lambda i: (i, 0))
    full = lambda a: pl.BlockSpec(a.shape, lambda i: (0,) * a.ndim)
    return pl.pallas_call(
        _in_mlp_body,
        grid=(n // ROW_BLK,),
        in_specs=[row_spec] + [full(a) for a in (w1, g1, b1, w2, g2, b2, wt, gt, bt)],
        out_specs=pl.BlockSpec((ROW_BLK, N_MAP), lambda i: (i, 0)),
        out_shape=jax.ShapeDtypeStruct((n, N_MAP), jnp.float32),
    )(feats, w1, g1, b1, w2, g2, b2, wt, gt, bt)


def _pre_body(x_ref, w_ref, g_ref, b_ref, o_ref):
    o_ref[...] = jax.nn.relu(
        _gn(jnp.dot(x_ref[...], w_ref[...], preferred_element_type=jnp.float32),
            g_ref[...], b_ref[...]))


def _pre(feat, w, g, b):
    n = feat.shape[0]
    full = lambda a: pl.BlockSpec(a.shape, lambda i: (0,) * a.ndim)
    return pl.pallas_call(
        _pre_body,
        grid=(n // ROW_BLK,),
        in_specs=[pl.BlockSpec((ROW_BLK, N_MAP), lambda i: (i, 0)),
                  full(w), full(g), full(b)],
        out_specs=pl.BlockSpec((ROW_BLK, N_MAP), lambda i: (i, 0)),
        out_shape=jax.ShapeDtypeStruct((n, N_MAP), jnp.float32),
    )(feat, w, g, b)


def _post_body(feat_ref, agg_ref, wa_ref, wb_ref, g2_ref, b2_ref,
               wl_ref, gl_ref, bl_ref, o_ref):
    feat = feat_ref[...]
    x = (jnp.dot(feat, wa_ref[...], preferred_element_type=jnp.float32)
         + jnp.dot(agg_ref[...], wb_ref[...], preferred_element_type=jnp.float32))
    x = jax.nn.relu(_gn(x, g2_ref[...], b2_ref[...]))
    x = _gn(jnp.dot(x, wl_ref[...], preferred_element_type=jnp.float32),
            gl_ref[...], bl_ref[...])
    o_ref[...] = jax.nn.relu(x + feat)


def _post(feat, agg, w2, g2, b2, wl, gl, bl):
    n = feat.shape[0]
    wa, wb = w2[:N_MAP], w2[N_MAP:]
    full = lambda a: pl.BlockSpec(a.shape, lambda i: (0,) * a.ndim)
    row = pl.BlockSpec((ROW_BLK, N_MAP), lambda i: (i, 0))
    return pl.pallas_call(
        _post_body,
        grid=(n // ROW_BLK,),
        in_specs=[row, row, full(wa), full(wb), full(g2), full(b2),
                  full(wl), full(gl), full(bl)],
        out_specs=row,
        out_shape=jax.ShapeDtypeStruct((n, N_MAP), jnp.float32),
    )(feat, agg, wa, wb, g2, b2, wl, gl, bl)


_MESH = plsc.VectorSubcoreMesh(core_axis_name="c", subcore_axis_name="s")


@functools.partial(
    pl.kernel,
    mesh=_MESH,
    out_type=jax.ShapeDtypeStruct((NT * NPT, N_MAP), jnp.float32),
    scratch_types=[pltpu.VMEM((48,), jnp.int32),
                   pltpu.VMEM((G,), jnp.int32),
                   pltpu.VMEM((G,), jnp.int32),
                   pltpu.VMEM((BUF_ROWS, N_MAP), jnp.float32),
                   pltpu.VMEM((G, N_MAP), jnp.float32),
                   pltpu.SemaphoreType.DMA],
)
def _agg_sc(ctx_hbm, us_hbm, vs_hbm, st_hbm, agg_hbm,
            sbuf, ub, vb, buf, rows, sem):
    w = lax.axis_index("s") * 2 + lax.axis_index("c")
    lo = w * NPT
    zf = jnp.zeros((16,), jnp.float32)

    def _zero(r, _):
        for c in range(N_MAP // 16):
            buf[r, pl.ds(c * 16, 16)] = zf
        return 0
    lax.fori_loop(0, BUF_ROWS, _zero, 0)

    pltpu.sync_copy(st_hbm, sbuf)
    se = sbuf[pl.ds(w, 16)]
    s16 = se[0] & ~15
    eal = (se[1] + 15) & ~15
    ng = (eal - s16 + G - 1) // G

    def _chunk(kg, _):
        base = pl.multiple_of(s16 + kg * G, 8)
        pltpu.sync_copy(us_hbm.at[pl.ds(base, G)], ub)
        pltpu.sync_copy(vs_hbm.at[pl.ds(base, G)], vb)
        pltpu.async_copy(ctx_hbm.at[ub], rows, sem).wait()

        def _grp(i, _):
            nbv = vb[pl.ds(i * 16, 16)] - lo
            ok = (nbv >= 0) & (nbv < NPT)
            nb16 = jnp.where(ok, nbv, NPT)
            for l in range(16):
                g = i * 16 + l
                nb = nb16[l]
                for c in range(N_MAP // 16):
                    sl = pl.ds(c * 16, 16)
                    buf[nb, sl] = jnp.maximum(buf[nb, sl], rows[g, sl])
            return 0
        lax.fori_loop(0, G // 16, _grp, 0)
        return 0
    lax.fori_loop(0, ng, _chunk, 0)

    off = pl.multiple_of(lo, 8)
    pltpu.sync_copy(buf.at[pl.ds(0, NPT)], agg_hbm.at[pl.ds(off, NPT)])


def kernel(feats, u, v, in_W1, in_g1, in_b1, in_W2, in_g2, in_b2, in_Wt,
           in_gt, in_bt, fc1_W, fc1_g, fc1_b, fc2_W, fc2_g, fc2_b,
           lin_W, lin_g, lin_b):
    feat = _input_mlp(feats, in_W1, in_g1, in_b1, in_W2, in_g2, in_b2,
                      in_Wt, in_gt, in_bt)
    n_scales = u.shape[0]
    n_blk = fc1_W.shape[0] // n_scales

    upad = (jnp.arange(G, dtype=jnp.int32) * 19) % N_NODES
    vpad = jnp.full((G,), NT * NPT, jnp.int32)
    bounds = jnp.arange(0, NT * NPT + 1, NPT, dtype=jnp.int32)
    edges = []
    for i in range(n_scales):
        vs, us = jax.lax.sort((v[i], u[i]), num_keys=1)
        starts = jnp.searchsorted(vs, bounds).astype(jnp.int32)
        starts = jnp.concatenate([starts, jnp.full((15,), E, jnp.int32)])
        us = jnp.concatenate([us, upad])
        vs = jnp.concatenate([vs, vpad])
        edges.append((us, vs, starts))

    for j in range(n_blk):
        for i in range(n_scales):
            t = j * n_scales + i
            ctx = _pre(feat, fc1_W[t], fc1_g[t], fc1_b[t])
            us, vs, starts = edges[i]
            agg = _agg_sc(ctx, us, vs, starts)[:N_NODES]
            feat = _post(feat, agg, fc2_W[t], fc2_g[t], fc2_b[t],
                         lin_W[t], lin_g[t], lin_b[t])
    return feat


# speedup vs baseline: 2.5803x; 1.1067x over previous
import functools

import jax
import jax.numpy as jnp
from jax import lax
from jax.experimental import pallas as pl
from jax.experimental.pallas import tpu as pltpu
from jax.experimental.pallas import tpu_sc as plsc

N_NODES = 10000
N_MAP = 128
E = 320000
ROW_BLK = 2000
_EPS = 1e-5

NT = 32
NPT = 320
BUF_ROWS = 328
G = 256
EP = E + 4 * G


def _gn(x, g, b):
    mu = jnp.mean(x, axis=1, keepdims=True)
    var = jnp.mean((x - mu) ** 2, axis=1, keepdims=True)
    return (x - mu) * jax.lax.rsqrt(var + _EPS) * g + b


def _in_mlp_body(x_ref, w1_ref, g1_ref, b1_ref, w2_ref, g2_ref, b2_ref,
                 wt_ref, gt_ref, bt_ref, o_ref):
    x = x_ref[...]
    h = jax.nn.relu(_gn(jnp.dot(x, w1_ref[...],
                                preferred_element_type=jnp.float32),
                        g1_ref[...], b1_ref[...]))
    h2 = _gn(jnp.dot(h, w2_ref[...], preferred_element_type=jnp.float32),
             g2_ref[...], b2_ref[...])
    t = _gn(jnp.dot(x, wt_ref[...], preferred_element_type=jnp.float32),
            gt_ref[...], bt_ref[...])
    o_ref[...] = jax.nn.relu(h2 + t)


def _input_mlp(feats, w1, g1, b1, w2, g2, b2, wt, gt, bt):
    n = feats.shape[0]
    row_spec = pl.BlockSpec((ROW_BLK, feats.shape[1]), lambda i: (i, 0))
    full = lambda a: pl.BlockSpec(a.shape, lambda i: (0,) * a.ndim)
    return pl.pallas_call(
        _in_mlp_body,
        grid=(n // ROW_BLK,),
        in_specs=[row_spec] + [full(a) for a in (w1, g1, b1, w2, g2, b2, wt, gt, bt)],
        out_specs=pl.BlockSpec((ROW_BLK, N_MAP), lambda i: (i, 0)),
        out_shape=jax.ShapeDtypeStruct((n, N_MAP), jnp.float32),
    )(feats, w1, g1, b1, w2, g2, b2, wt, gt, bt)


def _pre_body(x_ref, w_ref, g_ref, b_ref, o_ref):
    o_ref[...] = jax.nn.relu(
        _gn(jnp.dot(x_ref[...], w_ref[...], preferred_element_type=jnp.float32),
            g_ref[...], b_ref[...]))


def _pre(feat, w, g, b):
    n = feat.shape[0]
    full = lambda a: pl.BlockSpec(a.shape, lambda i: (0,) * a.ndim)
    return pl.pallas_call(
        _pre_body,
        grid=(n // ROW_BLK,),
        in_specs=[pl.BlockSpec((ROW_BLK, N_MAP), lambda i: (i, 0)),
                  full(w), full(g), full(b)],
        out_specs=pl.BlockSpec((ROW_BLK, N_MAP), lambda i: (i, 0)),
        out_shape=jax.ShapeDtypeStruct((n, N_MAP), jnp.float32),
    )(feat, w, g, b)


def _post_body(feat_ref, agg_ref, wa_ref, wb_ref, g2_ref, b2_ref,
               wl_ref, gl_ref, bl_ref, o_ref):
    feat = feat_ref[...]
    x = (jnp.dot(feat, wa_ref[...], preferred_element_type=jnp.float32)
         + jnp.dot(agg_ref[...], wb_ref[...], preferred_element_type=jnp.float32))
    x = jax.nn.relu(_gn(x, g2_ref[...], b2_ref[...]))
    x = _gn(jnp.dot(x, wl_ref[...], preferred_element_type=jnp.float32),
            gl_ref[...], bl_ref[...])
    o_ref[...] = jax.nn.relu(x + feat)


def _post(feat, agg, w2, g2, b2, wl, gl, bl):
    n = feat.shape[0]
    wa, wb = w2[:N_MAP], w2[N_MAP:]
    full = lambda a: pl.BlockSpec(a.shape, lambda i: (0,) * a.ndim)
    row = pl.BlockSpec((ROW_BLK, N_MAP), lambda i: (i, 0))
    return pl.pallas_call(
        _post_body,
        grid=(n // ROW_BLK,),
        in_specs=[row, row, full(wa), full(wb), full(g2), full(b2),
                  full(wl), full(gl), full(bl)],
        out_specs=row,
        out_shape=jax.ShapeDtypeStruct((n, N_MAP), jnp.float32),
    )(feat, agg, wa, wb, g2, b2, wl, gl, bl)


_MESH = plsc.VectorSubcoreMesh(core_axis_name="c", subcore_axis_name="s")


@functools.partial(
    pl.kernel,
    mesh=_MESH,
    out_type=jax.ShapeDtypeStruct((NT * NPT, N_MAP), jnp.float32),
    scratch_types=[pltpu.VMEM((48,), jnp.int32),
                   pltpu.VMEM((G,), jnp.int32),
                   pltpu.VMEM((G,), jnp.int32),
                   pltpu.VMEM((G,), jnp.int32),
                   pltpu.VMEM((G,), jnp.int32),
                   pltpu.VMEM((BUF_ROWS, N_MAP), jnp.float32),
                   pltpu.VMEM((G, N_MAP), jnp.float32),
                   pltpu.VMEM((G, N_MAP), jnp.float32),
                   pltpu.SemaphoreType.DMA,
                   pltpu.SemaphoreType.DMA,
                   pltpu.SemaphoreType.DMA,
                   pltpu.SemaphoreType.DMA],
)
def _agg_sc(ctx_hbm, us_hbm, vs_hbm, st_hbm, agg_hbm,
            sbuf, ub0, vb0, ub1, vb1, buf, rows0, rows1,
            ls0, ls1, gs0, gs1):
    w = lax.axis_index("s") * 2 + lax.axis_index("c")
    lo = w * NPT
    zf = jnp.zeros((16,), jnp.float32)

    def _zero(r, _):
        for c in range(N_MAP // 16):
            buf[r, pl.ds(c * 16, 16)] = zf
        return 0
    lax.fori_loop(0, BUF_ROWS, _zero, 0)

    pltpu.sync_copy(st_hbm, sbuf)
    se = sbuf[pl.ds(w, 16)]
    s16 = se[0] & ~15
    eal = (se[1] + 15) & ~15
    ng = (eal - s16 + G - 1) // G
    ng2 = ng + (ng & 1)

    slots = ((ub0, vb0, rows0, ls0, gs0), (ub1, vb1, rows1, ls1, gs1))

    def _lin(k, s):
        ub, vb, _, ls, _ = slots[s]
        base = pl.multiple_of(s16 + k * G, 8)
        pltpu.async_copy(us_hbm.at[pl.ds(base, G)], ub, ls)
        pltpu.async_copy(vs_hbm.at[pl.ds(base, G)], vb, ls)

    def _lin_wait(s):
        ub, vb, _, ls, _ = slots[s]
        pltpu.make_async_copy(us_hbm.at[pl.ds(0, G)], ub, ls).wait()
        pltpu.make_async_copy(vs_hbm.at[pl.ds(0, G)], vb, ls).wait()

    def _gat(s):
        ub, _, rows, _, gs = slots[s]
        pltpu.async_copy(ctx_hbm.at[ub], rows, gs)

    def _gat_wait(s):
        ub, _, rows, _, gs = slots[s]
        pltpu.make_async_copy(ctx_hbm.at[ub], rows, gs).wait()

    def _rmw(s):
        _, vb, rows, _, _ = slots[s]

        def _grp(i, _):
            nbv = vb[pl.ds(i * 16, 16)] - lo
            ok = (nbv >= 0) & (nbv < NPT)
            nb16 = jnp.where(ok, nbv, NPT)
            for l in range(16):
                g = i * 16 + l
                nb = nb16[l]
                for c in range(N_MAP // 16):
                    sl = pl.ds(c * 16, 16)
                    buf[nb, sl] = jnp.maximum(buf[nb, sl], rows[g, sl])
            return 0
        lax.fori_loop(0, G // 16, _grp, 0)

    _lin(0, 0)
    _lin_wait(0)
    _gat(0)
    _lin(1, 1)

    def _pair(p, _):
        a = 2 * p
        _gat_wait(0)
        _lin_wait(1)
        _gat(1)
        _rmw(0)
        _lin(a + 2, 0)
        _gat_wait(1)
        _lin_wait(0)
        _gat(0)
        _rmw(1)
        _lin(a + 3, 1)
        return 0
    lax.fori_loop(0, ng2 // 2, _pair, 0)

    _gat_wait(0)
    _lin_wait(1)

    off = pl.multiple_of(lo, 8)
    pltpu.sync_copy(buf.at[pl.ds(0, NPT)], agg_hbm.at[pl.ds(off, NPT)])


def kernel(feats, u, v, in_W1, in_g1, in_b1, in_W2, in_g2, in_b2, in_Wt,
           in_gt, in_bt, fc1_W, fc1_g, fc1_b, fc2_W, fc2_g, fc2_b,
           lin_W, lin_g, lin_b):
    feat = _input_mlp(feats, in_W1, in_g1, in_b1, in_W2, in_g2, in_b2,
                      in_Wt, in_gt, in_bt)
    n_scales = u.shape[0]
    n_blk = fc1_W.shape[0] // n_scales

    upad = (jnp.arange(EP - E, dtype=jnp.int32) * 19) % N_NODES
    vpad = jnp.full((EP - E,), NT * NPT, jnp.int32)
    bounds = jnp.arange(0, NT * NPT + 1, NPT, dtype=jnp.int32)
    edges = []
    for i in range(n_scales):
        vs, us = jax.lax.sort((v[i], u[i]), num_keys=1)
        starts = jnp.searchsorted(vs, bounds).astype(jnp.int32)
        starts = jnp.concatenate([starts, jnp.full((15,), E, jnp.int32)])
        us = jnp.concatenate([us, upad])
        vs = jnp.concatenate([vs, vpad])
        edges.append((us, vs, starts))

    for j in range(n_blk):
        for i in range(n_scales):
            t = j * n_scales + i
            ctx = _pre(feat, fc1_W[t], fc1_g[t], fc1_b[t])
            us, vs, starts = edges[i]
            agg = _agg_sc(ctx, us, vs, starts)[:N_NODES]
            feat = _post(feat, agg, fc2_W[t], fc2_g[t], fc2_b[t],
                         lin_W[t], lin_g[t], lin_b[t])
    return feat


# speedup vs baseline: 2.6103x; 1.0116x over previous
import functools

import jax
import jax.numpy as jnp
from jax import lax
from jax.experimental import pallas as pl
from jax.experimental.pallas import tpu as pltpu
from jax.experimental.pallas import tpu_sc as plsc

N_NODES = 10000
N_MAP = 128
E = 320000
ROW_BLK = 2000
_EPS = 1e-5

NT = 32
NPT = 320
BUF_ROWS = 328
G = 256
EP = E + 4 * G


def _gn(x, g, b):
    mu = jnp.mean(x, axis=1, keepdims=True)
    var = jnp.mean((x - mu) ** 2, axis=1, keepdims=True)
    return (x - mu) * jax.lax.rsqrt(var + _EPS) * g + b


def _in_mlp_body(x_ref, w1_ref, g1_ref, b1_ref, w2_ref, g2_ref, b2_ref,
                 wt_ref, gt_ref, bt_ref, o_ref):
    x = x_ref[...]
    h = jax.nn.relu(_gn(jnp.dot(x, w1_ref[...],
                                preferred_element_type=jnp.float32),
                        g1_ref[...], b1_ref[...]))
    h2 = _gn(jnp.dot(h, w2_ref[...], preferred_element_type=jnp.float32),
             g2_ref[...], b2_ref[...])
    t = _gn(jnp.dot(x, wt_ref[...], preferred_element_type=jnp.float32),
            gt_ref[...], bt_ref[...])
    o_ref[...] = jax.nn.relu(h2 + t)


def _input_mlp(feats, w1, g1, b1, w2, g2, b2, wt, gt, bt):
    n = feats.shape[0]
    row_spec = pl.BlockSpec((ROW_BLK, feats.shape[1]), lambda i: (i, 0))
    full = lambda a: pl.BlockSpec(a.shape, lambda i: (0,) * a.ndim)
    return pl.pallas_call(
        _in_mlp_body,
        grid=(n // ROW_BLK,),
        in_specs=[row_spec] + [full(a) for a in (w1, g1, b1, w2, g2, b2, wt, gt, bt)],
        out_specs=pl.BlockSpec((ROW_BLK, N_MAP), lambda i: (i, 0)),
        out_shape=jax.ShapeDtypeStruct((n, N_MAP), jnp.float32),
    )(feats, w1, g1, b1, w2, g2, b2, wt, gt, bt)


def _pre_body(x_ref, w_ref, g_ref, b_ref, o_ref):
    o_ref[...] = jax.nn.relu(
        _gn(jnp.dot(x_ref[...], w_ref[...], preferred_element_type=jnp.float32),
            g_ref[...], b_ref[...]))


def _pre(feat, w, g, b):
    n = feat.shape[0]
    full = lambda a: pl.BlockSpec(a.shape, lambda i: (0,) * a.ndim)
    return pl.pallas_call(
        _pre_body,
        grid=(n // ROW_BLK,),
        in_specs=[pl.BlockSpec((ROW_BLK, N_MAP), lambda i: (i, 0)),
                  full(w), full(g), full(b)],
        out_specs=pl.BlockSpec((ROW_BLK, N_MAP), lambda i: (i, 0)),
        out_shape=jax.ShapeDtypeStruct((n, N_MAP), jnp.float32),
    )(feat, w, g, b)


def _post_body(feat_ref, agg_ref, wa_ref, wb_ref, g2_ref, b2_ref,
               wl_ref, gl_ref, bl_ref, o_ref):
    feat = feat_ref[...]
    x = (jnp.dot(feat, wa_ref[...], preferred_element_type=jnp.float32)
         + jnp.dot(agg_ref[...], wb_ref[...], preferred_element_type=jnp.float32))
    x = jax.nn.relu(_gn(x, g2_ref[...], b2_ref[...]))
    x = _gn(jnp.dot(x, wl_ref[...], preferred_element_type=jnp.float32),
            gl_ref[...], bl_ref[...])
    o_ref[...] = jax.nn.relu(x + feat)


def _post(feat, agg, w2, g2, b2, wl, gl, bl):
    n = feat.shape[0]
    wa, wb = w2[:N_MAP], w2[N_MAP:]
    full = lambda a: pl.BlockSpec(a.shape, lambda i: (0,) * a.ndim)
    row = pl.BlockSpec((ROW_BLK, N_MAP), lambda i: (i, 0))
    return pl.pallas_call(
        _post_body,
        grid=(n // ROW_BLK,),
        in_specs=[row, row, full(wa), full(wb), full(g2), full(b2),
                  full(wl), full(gl), full(bl)],
        out_specs=row,
        out_shape=jax.ShapeDtypeStruct((n, N_MAP), jnp.float32),
    )(feat, agg, wa, wb, g2, b2, wl, gl, bl)


_MESH = plsc.VectorSubcoreMesh(core_axis_name="c", subcore_axis_name="s")


@functools.partial(
    pl.kernel,
    mesh=_MESH,
    out_type=jax.ShapeDtypeStruct((NT * NPT, N_MAP), jnp.float32),
    scratch_types=[pltpu.VMEM((48,), jnp.int32),
                   pltpu.VMEM((G,), jnp.int32),
                   pltpu.VMEM((G,), jnp.int32),
                   pltpu.VMEM((G,), jnp.int32),
                   pltpu.VMEM((G,), jnp.int32),
                   pltpu.VMEM((BUF_ROWS, N_MAP), jnp.float32),
                   pltpu.VMEM((G, N_MAP), jnp.float32),
                   pltpu.VMEM((G, N_MAP), jnp.float32),
                   pltpu.SemaphoreType.DMA,
                   pltpu.SemaphoreType.DMA,
                   pltpu.SemaphoreType.DMA,
                   pltpu.SemaphoreType.DMA],
)
def _agg_sc(ctx_hbm, us_hbm, vs_hbm, st_hbm, agg_hbm,
            sbuf, ub0, vb0, ub1, vb1, buf, rows0, rows1,
            ls0, ls1, gs0, gs1):
    w = lax.axis_index("s") * 2 + lax.axis_index("c")
    lo = w * NPT
    zf = jnp.zeros((16,), jnp.float32)

    def _zero(r, _):
        for c in range(N_MAP // 16):
            buf[r, pl.ds(c * 16, 16)] = zf
        return 0
    lax.fori_loop(0, BUF_ROWS, _zero, 0)

    pltpu.sync_copy(st_hbm, sbuf)
    se = sbuf[pl.ds(w, 16)]
    s16 = se[0] & ~255
    eal = (se[1] + 255) & ~255
    ng = (eal - s16 + G - 1) // G
    ng2 = ng + (ng & 1)

    slots = ((ub0, vb0, rows0, ls0, gs0), (ub1, vb1, rows1, ls1, gs1))

    def _lin(k, s):
        ub, vb, _, ls, _ = slots[s]
        base = pl.multiple_of(s16 + k * G, 8)
        pltpu.async_copy(us_hbm.at[pl.ds(base, G)], ub, ls)
        pltpu.async_copy(vs_hbm.at[pl.ds(base, G)], vb, ls)

    def _lin_wait(s):
        ub, vb, _, ls, _ = slots[s]
        pltpu.make_async_copy(us_hbm.at[pl.ds(0, G)], ub, ls).wait()
        pltpu.make_async_copy(vs_hbm.at[pl.ds(0, G)], vb, ls).wait()

    def _gat(s):
        ub, _, rows, _, gs = slots[s]
        pltpu.async_copy(ctx_hbm.at[ub], rows, gs)

    def _gat_wait(s):
        ub, _, rows, _, gs = slots[s]
        pltpu.make_async_copy(ctx_hbm.at[ub], rows, gs).wait()

    def _rmw(s):
        _, vb, rows, _, _ = slots[s]

        def _grp(i, _):
            nbv = vb[pl.ds(i * 16, 16)] - lo
            ok = (nbv >= 0) & (nbv < NPT)
            nb16 = jnp.where(ok, nbv, NPT)
            for l in range(16):
                g = i * 16 + l
                nb = nb16[l]
                for c in range(N_MAP // 16):
                    sl = pl.ds(c * 16, 16)
                    buf[nb, sl] = jnp.maximum(buf[nb, sl], rows[g, sl])
            return 0
        lax.fori_loop(0, G // 16, _grp, 0)

    _lin(0, 0)
    _lin_wait(0)
    _gat(0)
    _lin(1, 1)

    def _pair(p, _):
        a = 2 * p
        _gat_wait(0)
        _lin_wait(1)
        _gat(1)
        _rmw(0)
        _lin(a + 2, 0)
        _gat_wait(1)
        _lin_wait(0)
        _gat(0)
        _rmw(1)
        _lin(a + 3, 1)
        return 0
    lax.fori_loop(0, ng2 // 2, _pair, 0)

    _gat_wait(0)
    _lin_wait(1)

    off = pl.multiple_of(lo, 8)
    pltpu.sync_copy(buf.at[pl.ds(0, NPT)], agg_hbm.at[pl.ds(off, NPT)])


def kernel(feats, u, v, in_W1, in_g1, in_b1, in_W2, in_g2, in_b2, in_Wt,
           in_gt, in_bt, fc1_W, fc1_g, fc1_b, fc2_W, fc2_g, fc2_b,
           lin_W, lin_g, lin_b):
    feat = _input_mlp(feats, in_W1, in_g1, in_b1, in_W2, in_g2, in_b2,
                      in_Wt, in_gt, in_bt)
    n_scales = u.shape[0]
    n_blk = fc1_W.shape[0] // n_scales

    upad = (jnp.arange(EP - E, dtype=jnp.int32) * 19) % N_NODES
    vpad = jnp.full((EP - E,), NT * NPT, jnp.int32)
    bounds = jnp.arange(0, NT * NPT + 1, NPT, dtype=jnp.int32)
    edges = []
    for i in range(n_scales):
        ks = jax.lax.sort((v[i] << 14) | u[i])
        vs = ks >> 14
        us = ks & 16383
        starts = jnp.searchsorted(vs, bounds).astype(jnp.int32)
        starts = jnp.concatenate([starts, jnp.full((15,), E, jnp.int32)])
        us = jnp.concatenate([us, upad]).reshape(-1, 16, 16)
        vs = jnp.concatenate([vs, vpad]).reshape(-1, 16, 16)
        us = us.swapaxes(1, 2).reshape(-1)
        vs = vs.swapaxes(1, 2).reshape(-1)
        edges.append((us, vs, starts))

    for j in range(n_blk):
        for i in range(n_scales):
            t = j * n_scales + i
            ctx = _pre(feat, fc1_W[t], fc1_g[t], fc1_b[t])
            us, vs, starts = edges[i]
            agg = _agg_sc(ctx, us, vs, starts)[:N_NODES]
            feat = _post(feat, agg, fc2_W[t], fc2_g[t], fc2_b[t],
                         lin_W[t], lin_g[t], lin_b[t])
    return feat
